# baseline (speedup 1.0000x reference)
import functools

import jax
import jax.numpy as jnp
from jax import lax
from jax.experimental import pallas as pl
from jax.experimental.pallas import tpu as pltpu
from jax.experimental.pallas import tpu_sc as plsc

_LANES = 16
_STRIDE = 1024


def _interaction_table_kernel(cw_ref, sw_ref, cb_ref, sb_ref, g_ref):
    g = lax.dot_general(
        cw_ref[...],
        sw_ref[...],
        (((1,), (1,)), ((), ())),
        preferred_element_type=jnp.float32,
        precision=lax.Precision.HIGHEST,
    )
    g_ref[...] = g + cb_ref[...] + sb_ref[...]


def _build_interaction_table(c_weight, s_weight, c_biase, s_biase):
    v, d = c_weight.shape
    sw_pad = jnp.zeros((_STRIDE, d), jnp.float32).at[:v].set(s_weight)
    sb_pad = jnp.zeros((1, _STRIDE), jnp.float32).at[:, :v].set(
        s_biase.reshape(1, v))
    return pl.pallas_call(
        _interaction_table_kernel,
        out_shape=jax.ShapeDtypeStruct((v, _STRIDE), jnp.float32),
    )(c_weight, sw_pad, c_biase, sb_pad)


def _make_sc_gather(b, num_workers, chunk):
    per_w = b // num_workers
    rows = per_w // chunk
    mesh = plsc.VectorSubcoreMesh(core_axis_name="c", subcore_axis_name="s")

    @functools.partial(
        pl.kernel,
        mesh=mesh,
        out_type=jax.ShapeDtypeStruct((b,), jnp.float32),
        scratch_types=[
            pltpu.VMEM((per_w,), jnp.int32),
            pltpu.VMEM((per_w,), jnp.int32),
            pltpu.VMEM((rows, chunk), jnp.int32),
            pltpu.VMEM((per_w,), jnp.float32),
            pltpu.SemaphoreType.DMA,
        ],
    )
    def sc_gather(g_hbm, c_hbm, s_hbm, out_hbm, c_v, s_v, idx_v, val_v, sem):
        wid = lax.axis_index("s") * 2 + lax.axis_index("c")
        base = wid * per_w
        pltpu.sync_copy(c_hbm.at[pl.ds(base, per_w)], c_v)
        pltpu.sync_copy(s_hbm.at[pl.ds(base, per_w)], s_v)
        for r in range(rows):
            for i in range(chunk // _LANES):
                sl = pl.ds(r * chunk + i * _LANES, _LANES)
                idx_v[r, pl.ds(i * _LANES, _LANES)] = (
                    c_v[sl] * _STRIDE + s_v[sl])
        copies = [
            pltpu.async_copy(
                g_hbm.at[idx_v.at[r]], val_v.at[pl.ds(r * chunk, chunk)], sem)
            for r in range(rows)
        ]
        for cp in copies:
            cp.wait()
        pltpu.sync_copy(val_v, out_hbm.at[pl.ds(base, per_w)])

    return sc_gather


def kernel(c, s, c_weight, c_biase, s_weight, s_biase):
    v, _ = c_weight.shape
    b = c.shape[0]

    g = _build_interaction_table(c_weight, s_weight, c_biase, s_biase)
    g_flat = g.reshape(v * _STRIDE)

    out = _make_sc_gather(b, 32, 128)(
        g_flat, c.astype(jnp.int32), s.astype(jnp.int32))
    return out.reshape(b, 1)


# speedup vs baseline: 1.0859x; 1.0859x over previous
import functools

import jax
import jax.numpy as jnp
from jax import lax
from jax.experimental import pallas as pl
from jax.experimental.pallas import tpu as pltpu
from jax.experimental.pallas import tpu_sc as plsc

_LANES = 16


def _interaction_table_kernel(cw_ref, sw_ref, cb_ref, sb_ref, g_ref):
    g = lax.dot_general(
        cw_ref[...],
        sw_ref[...],
        (((1,), (1,)), ((), ())),
        preferred_element_type=jnp.float32,
        precision=lax.Precision.HIGHEST,
    )
    g_ref[...] = g + cb_ref[...] + sb_ref[...]


def _build_interaction_table(c_weight, s_weight, c_biase, s_biase):
    v = c_weight.shape[0]
    return pl.pallas_call(
        _interaction_table_kernel,
        out_shape=jax.ShapeDtypeStruct((v, v), jnp.float32),
    )(c_weight, s_weight, c_biase, s_biase.reshape(1, v))


def _make_sc_gather(v, b, num_workers, chunk):
    per_w = b // num_workers
    rows = per_w // chunk
    mesh = plsc.VectorSubcoreMesh(core_axis_name="c", subcore_axis_name="s")

    @functools.partial(
        pl.kernel,
        mesh=mesh,
        out_type=jax.ShapeDtypeStruct((b,), jnp.float32),
        scratch_types=[
            pltpu.VMEM((per_w,), jnp.int32),
            pltpu.VMEM((per_w,), jnp.int32),
            pltpu.VMEM((per_w,), jnp.int32),
            pltpu.VMEM((per_w,), jnp.float32),
            pltpu.SemaphoreType.DMA,
        ],
    )
    def sc_gather(g_hbm, c_hbm, s_hbm, out_hbm, c_v, s_v, idx_v, val_v, sem):
        wid = lax.axis_index("s") * 2 + lax.axis_index("c")
        base = wid * per_w
        pltpu.sync_copy(c_hbm.at[pl.ds(base, per_w)], c_v)
        pltpu.sync_copy(s_hbm.at[pl.ds(base, per_w)], s_v)

        def body(i, carry):
            sl = pl.ds(i * _LANES, _LANES)
            idx_v[sl] = c_v[sl] * v + s_v[sl]
            return carry

        lax.fori_loop(0, per_w // _LANES, body, 0)
        copies = [
            pltpu.async_copy(
                g_hbm.at[idx_v.at[pl.ds(r * chunk, chunk)]],
                val_v.at[pl.ds(r * chunk, chunk)],
                sem,
            )
            for r in range(rows)
        ]
        for cp in copies:
            cp.wait()
        pltpu.sync_copy(val_v, out_hbm.at[pl.ds(base, per_w)])

    return sc_gather


def kernel(c, s, c_weight, c_biase, s_weight, s_biase):
    v, _ = c_weight.shape
    b = c.shape[0]

    g = _build_interaction_table(c_weight, s_weight, c_biase, s_biase)
    g_flat = g.reshape(v * v)

    out = _make_sc_gather(v, b, 32, 128)(
        g_flat, c.astype(jnp.int32), s.astype(jnp.int32))
    return out.reshape(b, 1)
